# Initial kernel scaffold; baseline (speedup 1.0000x reference)
import jax
import jax.numpy as jnp
from jax.experimental import pallas as pl


def kernel(w1, b1, w2, b2, g1, g2, fc1_w, fc1_b, mfc1_w, mfc1_b, mfc2_w, mfc2_b, fcw_cnn, fcw_mlp, fc_b, x_cnn, x_mlp):
    raise NotImplementedError("write your pallas kernel here")


# speedup vs baseline: 1.6933x; 1.6933x over previous
import jax
import jax.numpy as jnp
from jax.experimental import pallas as pl
from jax.experimental.pallas import tpu as pltpu

H, W = 40, 28
HP, WP = H + 2, W + 2
SX = 1408
A1W = 1344
NR1 = 1280
NP1 = 392
NSP1 = 352
NR2 = 320
NOUT = 128
K2 = 64 * NOUT

G = 16
CH = 2
XW = G * SX + 64
BT2 = 256


def _rup(n, m):
    return ((n + m - 1) // m) * m


def _conv_kernel(x_ref, w1_ref, b1_ref, w2_ref, b2_ref, g1_ref, g2_ref,
                 o_ref, m1s, rbuf, m2s):
    bf = jnp.bfloat16
    f32 = jnp.float32
    w1 = w1_ref[...]
    b1 = b1_ref[...]
    xv = x_ref[0]

    CW = CH * SX
    PW = CW - 64
    for c in range(G // CH):
        x0 = c * CW
        acc = w1[0] * xv[:, x0:x0 + CW]
        for kh in range(3):
            for kw in range(3):
                if kh == 0 and kw == 0:
                    continue
                off = kh * WP + kw
                acc = acc + w1[kh * 3 + kw] * xv[:, x0 + off:x0 + off + CW]
        hm = jnp.maximum(jnp.maximum(acc[:, 0:PW], acc[:, 1:1 + PW]),
                         jnp.maximum(acc[:, WP:WP + PW],
                                     acc[:, WP + 1:WP + 1 + PW]))
        m1 = jnp.maximum(hm + b1, 0)
        m1s[64 * c:64 * c + 32, :] = m1[:, 0:NR1].astype(bf)
        m1s[64 * c + 32:64 * c + 64, :] = m1[:, SX:SX + NR1].astype(bf)

    p1 = jnp.dot(m1s[...], g1_ref[...], preferred_element_type=f32)
    p1b = p1.astype(bf)

    for t in range(9):
        off = (t // 3) * 16 + (t % 3)
        for g in range(G):
            rbuf[32 * t:32 * t + 32, NSP1 * g:NSP1 * (g + 1)] = (
                p1b[32 * g:32 * g + 32, off:off + NSP1])

    h2 = jnp.dot(w2_ref[...], rbuf[...], preferred_element_type=f32)
    W2L = G * NSP1 - 17
    hm2 = jnp.maximum(jnp.maximum(h2[:, 0:W2L], h2[:, 1:1 + W2L]),
                      jnp.maximum(h2[:, 16:16 + W2L], h2[:, 17:17 + W2L]))
    m2 = jnp.maximum(hm2 + b2_ref[...], 0)
    for g in range(G):
        m2s[64 * g:64 * (g + 1), :] = m2[:, NSP1 * g:NSP1 * g + NR2]

    feat = jnp.dot(m2s[...], g2_ref[...].astype(f32), preferred_element_type=f32)
    o_ref[0] = feat


def _conv_stack(xp, w1b, b1b, w2b, b2, g1b, g2b):
    NB = xp.shape[0]
    return pl.pallas_call(
        _conv_kernel,
        out_shape=jax.ShapeDtypeStruct((NB, G * 64, NOUT), jnp.float32),
        grid=(NB,),
        in_specs=[
            pl.BlockSpec((1, 1, XW), lambda i: (i, 0, 0)),
            pl.BlockSpec((9, 32, 1), lambda i: (0, 0, 0)),
            pl.BlockSpec((32, 1), lambda i: (0, 0)),
            pl.BlockSpec((64, 288), lambda i: (0, 0)),
            pl.BlockSpec((64, 1), lambda i: (0, 0)),
            pl.BlockSpec((NR1, NP1), lambda i: (0, 0)),
            pl.BlockSpec((NR2, NOUT), lambda i: (0, 0)),
        ],
        out_specs=pl.BlockSpec((1, G * 64, NOUT), lambda i: (i, 0, 0)),
        scratch_shapes=[
            pltpu.VMEM((G * 32, NR1), jnp.bfloat16),
            pltpu.VMEM((288, G * NSP1), jnp.bfloat16),
            pltpu.VMEM((G * 64, NR2), jnp.float32),
        ],
        compiler_params=pltpu.CompilerParams(
            dimension_semantics=("parallel",),
            vmem_limit_bytes=48 << 20),
    )(xp, w1b, b1b, w2b, b2, g1b, g2b)


def _head_kernel(z_ref, xm_ref, fc1w_ref, fc1b_ref, m1w_ref, m1b_ref,
                 m2w_ref, m2b_ref, fwc_ref, fwm_ref, fb_ref, o_ref):
    f32 = jnp.float32
    cnn = jnp.maximum(
        jnp.dot(z_ref[...], fc1w_ref[...], preferred_element_type=f32)
        + fc1b_ref[...], 0.0)
    m = jnp.maximum(
        jnp.dot(xm_ref[...], m1w_ref[...], preferred_element_type=f32)
        + m1b_ref[...], 0.0)
    m = jnp.maximum(
        jnp.dot(m, m2w_ref[...], preferred_element_type=f32)
        + m2b_ref[...], 0.0)
    o_ref[...] = (jnp.dot(cnn, fwc_ref[...], preferred_element_type=f32)
                  + jnp.dot(m, fwm_ref[...], preferred_element_type=f32)
                  + fb_ref[...])


def _head(z, xm, fc1w, fc1b, m1w, m1b, m2w, m2b, fwc, fwm, fb):
    B = z.shape[0]
    IN = xm.shape[1]

    def const(shape):
        return pl.BlockSpec(shape, lambda i: (0,) * len(shape))

    return pl.pallas_call(
        _head_kernel,
        out_shape=jax.ShapeDtypeStruct((B, 1), jnp.float32),
        grid=(B // BT2,),
        in_specs=[
            pl.BlockSpec((BT2, K2), lambda i: (i, 0)),
            pl.BlockSpec((BT2, IN), lambda i: (i, 0)),
            const((K2, 128)), const((1, 128)),
            const((IN, 64)), const((1, 64)),
            const((64, 96)), const((1, 96)),
            const((128, 1)), const((96, 1)), const((1, 1)),
        ],
        out_specs=pl.BlockSpec((BT2, 1), lambda i: (i, 0)),
        compiler_params=pltpu.CompilerParams(
            dimension_semantics=("parallel",),
            vmem_limit_bytes=48 << 20),
    )(z, xm, fc1w, fc1b, m1w, m1b, m2w, m2b, fwc, fwm, fb)


def kernel(w1, b1, w2, b2, g1, g2, fc1_w, fc1_b, mfc1_w, mfc1_b,
           mfc2_w, mfc2_b, fcw_cnn, fcw_mlp, fc_b, x_cnn, x_mlp):
    bf = jnp.bfloat16
    B = x_cnn.shape[0]
    BP = _rup(B, G)

    xp = jnp.pad(x_cnn[:, 0], ((0, BP - B), (1, 1), (1, 1)))
    xp = xp.reshape(BP, HP * WP)
    xp = jnp.pad(xp, ((0, 0), (0, SX - HP * WP)))
    xp = xp.reshape(BP // G, G * SX)
    xp = jnp.pad(xp, ((0, 0), (0, XW - G * SX)))[:, None, :]

    feat = _conv_stack(xp, w1, b1, w2.astype(bf),
                       b2, g1.astype(bf), g2.astype(bf))
    z = feat.reshape(BP, K2)

    BP2 = _rup(BP, BT2)
    z = jnp.pad(z, ((0, BP2 - BP), (0, 0)))
    xm = jnp.pad(x_mlp, ((0, BP2 - B), (0, 0)))
    out = _head(z, xm, fc1_w, fc1_b, mfc1_w, mfc1_b,
                mfc2_w, mfc2_b, fcw_cnn, fcw_mlp, fc_b)
    return out[:B]
